# Initial kernel scaffold; baseline (speedup 1.0000x reference)
import jax
import jax.numpy as jnp
from jax.experimental import pallas as pl


def kernel(valences, embed_table, device):
    raise NotImplementedError("write your pallas kernel here")


# speedup vs baseline: 4.4189x; 4.4189x over previous
import functools

import jax
import jax.numpy as jnp
from jax import lax
from jax.experimental import pallas as pl
from jax.experimental.pallas import tpu as pltpu
from jax.experimental.pallas import tpu_sc as plsc

_MAX_VALENCE = 6
_NUM_TYPES = 4
_EMBED = 64
_BATCH = 16384
_ATOMS = 50
_B = _BATCH * _ATOMS

_NC = 2
_NS = 16
_NW = _NC * _NS
_B_PER_W = _B // _NW
_CHUNK = 512
_N_CHUNKS = _B_PER_W // _CHUNK
_IDX_DMA = 128


def _make_kernel():
  mesh = plsc.VectorSubcoreMesh(core_axis_name="c", subcore_axis_name="s")

  @functools.partial(
      pl.kernel,
      mesh=mesh,
      compiler_params=pltpu.CompilerParams(use_tc_tiling_on_sc=False),
      out_type=jax.ShapeDtypeStruct((_B, _EMBED), jnp.float32),
      scratch_types=[
          pltpu.VMEM((_NUM_TYPES * _CHUNK,), jnp.int32),
          pltpu.VMEM((_CHUNK,), jnp.int32),
          pltpu.VMEM((_CHUNK, _EMBED), jnp.float32),
          pltpu.SemaphoreType.DMA,
      ],
  )
  def lookup(val_hbm, table_hbm, out_hbm, val_v, idx_v, rows_v, sem):
    wid = lax.axis_index("s") * _NC + lax.axis_index("c")
    base = wid * _B_PER_W

    def chunk_body(c, carry):
      off = base + c * _CHUNK
      for j in range(_NUM_TYPES):
        pltpu.sync_copy(val_hbm.at[pl.ds(j * _B + off, _CHUNK)],
                        val_v.at[pl.ds(j * _CHUNK, _CHUNK)])
      for g in range(_CHUNK // 16):
        gbase = g * 16
        v0 = val_v[pl.ds(gbase, 16)]
        v1 = val_v[pl.ds(_CHUNK + gbase, 16)]
        v2 = val_v[pl.ds(2 * _CHUNK + gbase, 16)]
        v3 = val_v[pl.ds(3 * _CHUNK + gbase, 16)]
        idx = v0 + v1 * 6 + v2 * 36 + v3 * 216
        idx_v[pl.ds(gbase, 16)] = idx
      for k in range(_CHUNK // _IDX_DMA):
        pltpu.async_copy(
            table_hbm.at[idx_v.at[pl.ds(k * _IDX_DMA, _IDX_DMA)]],
            rows_v.at[pl.ds(k * _IDX_DMA, _IDX_DMA)],
            sem,
        ).wait()
      pltpu.sync_copy(rows_v, out_hbm.at[pl.ds(off, _CHUNK)])
      return carry

    lax.fori_loop(0, _N_CHUNKS, chunk_body, 0)

  return lookup


_LOOKUP = _make_kernel()


def kernel(valences, embed_table, device):
  val_t = jnp.transpose(valences.reshape(_B, _NUM_TYPES)).reshape(-1)
  out = _LOOKUP(val_t, embed_table)
  return out.reshape(_BATCH, _ATOMS, _EMBED)


# speedup vs baseline: 5.2087x; 1.1787x over previous
import functools

import jax
import jax.numpy as jnp
from jax import lax
from jax.experimental import pallas as pl
from jax.experimental.pallas import tpu as pltpu
from jax.experimental.pallas import tpu_sc as plsc

_MAX_VALENCE = 6
_NUM_TYPES = 4
_EMBED = 64
_BATCH = 16384
_ATOMS = 50
_B = _BATCH * _ATOMS

_NC = 2
_NS = 16
_NW = _NC * _NS
_B_PER_W = _B // _NW
_CHUNK = 640
_N_CHUNKS = _B_PER_W // _CHUNK
_IDX_DMA = 128
_N_GATHERS = _CHUNK // _IDX_DMA


def _make_kernel():
  mesh = plsc.VectorSubcoreMesh(core_axis_name="c", subcore_axis_name="s")

  @functools.partial(
      pl.kernel,
      mesh=mesh,
      compiler_params=pltpu.CompilerParams(use_tc_tiling_on_sc=False),
      out_type=jax.ShapeDtypeStruct((_B, _EMBED), jnp.float32),
      scratch_types=[
          pltpu.VMEM((_NUM_TYPES * _CHUNK,), jnp.int32),
          pltpu.VMEM((_NUM_TYPES * _CHUNK,), jnp.int32),
          pltpu.VMEM((_CHUNK,), jnp.int32),
          pltpu.VMEM((_CHUNK,), jnp.int32),
          pltpu.VMEM((_CHUNK, _EMBED), jnp.float32),
          pltpu.VMEM((_CHUNK, _EMBED), jnp.float32),
          pltpu.SemaphoreType.DMA,
          pltpu.SemaphoreType.DMA,
          pltpu.SemaphoreType.DMA,
          pltpu.SemaphoreType.DMA,
          pltpu.SemaphoreType.DMA,
          pltpu.SemaphoreType.DMA,
      ],
  )
  def lookup(val_hbm, table_hbm, out_hbm,
             val0, val1, idx0, idx1, rows0, rows1,
             sv0, sv1, sg0, sg1, sw0, sw1):
    wid = lax.axis_index("s") * _NC + lax.axis_index("c")
    vals = (val0, val1)
    idxs = (idx0, idx1)
    rows = (rows0, rows1)
    sv = (sv0, sv1)
    sg = (sg0, sg1)
    sw = (sw0, sw1)
    vwords = _NUM_TYPES * _CHUNK

    def vin(c, b):
      off = (wid * _N_CHUNKS + c) * vwords
      return pltpu.make_async_copy(val_hbm.at[pl.ds(off, vwords)], vals[b],
                                   sv[b])

    def wout(c, b):
      off = wid * _B_PER_W + c * _CHUNK
      return pltpu.make_async_copy(rows[b], out_hbm.at[pl.ds(off, _CHUNK)],
                                   sw[b])

    def gth(b, k):
      return pltpu.make_async_copy(
          table_hbm.at[idxs[b].at[pl.ds(k * _IDX_DMA, _IDX_DMA)]],
          rows[b].at[pl.ds(k * _IDX_DMA, _IDX_DMA)],
          sg[b])

    vin(0, 0).start()
    vin(1, 1).start()

    def pair_body(i, carry):
      for b in range(2):
        c = 2 * i + b
        vin(c, b).wait()
        for g in range(_CHUNK // 16):
          gb = g * 16
          v0 = vals[b][pl.ds(gb, 16)]
          v1 = vals[b][pl.ds(_CHUNK + gb, 16)]
          v2 = vals[b][pl.ds(2 * _CHUNK + gb, 16)]
          v3 = vals[b][pl.ds(3 * _CHUNK + gb, 16)]
          idxs[b][pl.ds(gb, 16)] = v0 + v1 * 6 + v2 * 36 + v3 * 216
        @pl.when(c + 2 < _N_CHUNKS)
        def _prefetch():
          vin(c + 2, b).start()

        @pl.when(c >= 2)
        def _drain_prev_write():
          wout(c - 2, b).wait()
        for k in range(_N_GATHERS):
          gth(b, k).start()
        for k in range(_N_GATHERS):
          gth(b, k).wait()
        wout(c, b).start()
      return carry

    lax.fori_loop(0, _N_CHUNKS // 2, pair_body, 0)
    wout(_N_CHUNKS - 2, 0).wait()
    wout(_N_CHUNKS - 1, 1).wait()

  return lookup


_LOOKUP = _make_kernel()


def kernel(valences, embed_table, device):
  val_t = (valences.reshape(_NW, _N_CHUNKS, _CHUNK, _NUM_TYPES)
           .transpose(0, 1, 3, 2).reshape(-1))
  out = _LOOKUP(val_t, embed_table)
  return out.reshape(_BATCH, _ATOMS, _EMBED)


# speedup vs baseline: 5.3082x; 1.0191x over previous
import functools

import jax
import jax.numpy as jnp
from jax import lax
from jax.experimental import pallas as pl
from jax.experimental.pallas import tpu as pltpu
from jax.experimental.pallas import tpu_sc as plsc

_MAX_VALENCE = 6
_NUM_TYPES = 4
_EMBED = 64
_BATCH = 16384
_ATOMS = 50
_B = _BATCH * _ATOMS

_NC = 2
_NS = 16
_NW = _NC * _NS
_B_PER_W = _B // _NW
_CHUNK = 640
_N_CHUNKS = _B_PER_W // _CHUNK
_IDX_DMA = 128
_N_GATHERS = _CHUNK // _IDX_DMA


def _make_kernel():
  mesh = plsc.VectorSubcoreMesh(core_axis_name="c", subcore_axis_name="s")

  @functools.partial(
      pl.kernel,
      mesh=mesh,
      compiler_params=pltpu.CompilerParams(use_tc_tiling_on_sc=False),
      out_type=jax.ShapeDtypeStruct((_B, _EMBED), jnp.float32),
      scratch_types=[
          pltpu.VMEM((_CHUNK,), jnp.int32),
          pltpu.VMEM((_CHUNK,), jnp.int32),
          pltpu.VMEM((_CHUNK,), jnp.int32),
          pltpu.VMEM((_CHUNK,), jnp.int32),
          pltpu.VMEM((_CHUNK, _EMBED), jnp.float32),
          pltpu.VMEM((_CHUNK, _EMBED), jnp.float32),
          pltpu.SemaphoreType.DMA,
          pltpu.SemaphoreType.DMA,
          pltpu.SemaphoreType.DMA,
          pltpu.SemaphoreType.DMA,
          pltpu.SemaphoreType.DMA,
          pltpu.SemaphoreType.DMA,
      ],
  )
  def lookup(val_hbm, table_hbm, out_hbm,
             val0, val1, idx0, idx1, rows0, rows1,
             sv0, sv1, sg0, sg1, sw0, sw1):
    wid = lax.axis_index("s") * _NC + lax.axis_index("c")
    vals = (val0, val1)
    idxs = (idx0, idx1)
    rows = (rows0, rows1)
    sv = (sv0, sv1)
    sg = (sg0, sg1)
    sw = (sw0, sw1)

    def vin(c, b):
      off = wid * _B_PER_W + c * _CHUNK
      return pltpu.make_async_copy(val_hbm.at[pl.ds(off, _CHUNK)], vals[b],
                                   sv[b])

    def wout(c, b):
      off = wid * _B_PER_W + c * _CHUNK
      return pltpu.make_async_copy(rows[b], out_hbm.at[pl.ds(off, _CHUNK)],
                                   sw[b])

    def gth(b, k):
      return pltpu.make_async_copy(
          table_hbm.at[idxs[b].at[pl.ds(k * _IDX_DMA, _IDX_DMA)]],
          rows[b].at[pl.ds(k * _IDX_DMA, _IDX_DMA)],
          sg[b])

    vin(0, 0).start()
    vin(1, 1).start()

    def pair_body(i, carry):
      for b in range(2):
        c = 2 * i + b
        vin(c, b).wait()
        for g in range(_CHUNK // 16):
          gb = g * 16
          w = vals[b][pl.ds(gb, 16)]
          d0 = w & 255
          d1 = (w >> 8) & 255
          d2 = (w >> 16) & 255
          d3 = w >> 24
          idxs[b][pl.ds(gb, 16)] = d0 + d1 * 6 + d2 * 36 + d3 * 216
        @pl.when(c + 2 < _N_CHUNKS)
        def _prefetch():
          vin(c + 2, b).start()

        @pl.when(c >= 2)
        def _drain_prev_write():
          wout(c - 2, b).wait()
        for k in range(_N_GATHERS):
          gth(b, k).start()
        for k in range(_N_GATHERS):
          gth(b, k).wait()
        wout(c, b).start()
      return carry

    lax.fori_loop(0, _N_CHUNKS // 2, pair_body, 0)
    wout(_N_CHUNKS - 2, 0).wait()
    wout(_N_CHUNKS - 1, 1).wait()

  return lookup


_LOOKUP = _make_kernel()


def kernel(valences, embed_table, device):
  val_packed = lax.bitcast_convert_type(
      valences.reshape(_B, _NUM_TYPES).astype(jnp.int8), jnp.int32)
  out = _LOOKUP(val_packed, embed_table)
  return out.reshape(_BATCH, _ATOMS, _EMBED)


# speedup vs baseline: 6.9618x; 1.3115x over previous
import functools

import jax
import jax.numpy as jnp
from jax import lax
from jax.experimental import pallas as pl
from jax.experimental.pallas import tpu as pltpu
from jax.experimental.pallas import tpu_sc as plsc

_MAX_VALENCE = 6
_NUM_TYPES = 4
_VOCAB = _MAX_VALENCE ** _NUM_TYPES
_EMBED = 64
_BATCH = 16384
_ATOMS = 50
_B = _BATCH * _ATOMS

_NC = 2
_NS = 16
_NW = _NC * _NS
_B_PER_W = _B // _NW
_CHUNK = 640
_N_CHUNKS = _B_PER_W // _CHUNK
_IDX_DMA = 128
_N_GATHERS = _CHUNK // _IDX_DMA


def _make_kernel():
  mesh = plsc.VectorSubcoreMesh(core_axis_name="c", subcore_axis_name="s")

  @functools.partial(
      pl.kernel,
      mesh=mesh,
      compiler_params=pltpu.CompilerParams(use_tc_tiling_on_sc=False),
      out_type=jax.ShapeDtypeStruct((_B, _EMBED), jnp.float32),
      scratch_types=[
          pltpu.VMEM((_CHUNK,), jnp.int32),
          pltpu.VMEM((_CHUNK,), jnp.int32),
          pltpu.VMEM((_CHUNK,), jnp.int32),
          pltpu.VMEM((_CHUNK,), jnp.int32),
          pltpu.VMEM((_CHUNK, _EMBED), jnp.float32),
          pltpu.VMEM((_CHUNK, _EMBED), jnp.float32),
          pltpu.VMEM_SHARED((_VOCAB, _EMBED), jnp.float32),
          pltpu.SemaphoreType.DMA,
          pltpu.SemaphoreType.DMA,
          pltpu.SemaphoreType.DMA,
          pltpu.SemaphoreType.DMA,
          pltpu.SemaphoreType.DMA,
          pltpu.SemaphoreType.DMA,
      ],
  )
  def lookup(val_hbm, table_hbm, out_hbm,
             val0, val1, idx0, idx1, rows0, rows1, table_sh,
             sv0, sv1, sg0, sg1, sw0, sw1):
    sid = lax.axis_index("s")
    wid = sid * _NC + lax.axis_index("c")

    @pl.when(sid == 0)
    def _stage_table():
      pltpu.sync_copy(table_hbm, table_sh)

    plsc.subcore_barrier()
    vals = (val0, val1)
    idxs = (idx0, idx1)
    rows = (rows0, rows1)
    sv = (sv0, sv1)
    sg = (sg0, sg1)
    sw = (sw0, sw1)

    def vin(c, b):
      off = wid * _B_PER_W + c * _CHUNK
      return pltpu.make_async_copy(val_hbm.at[pl.ds(off, _CHUNK)], vals[b],
                                   sv[b])

    def wout(c, b):
      off = wid * _B_PER_W + c * _CHUNK
      return pltpu.make_async_copy(rows[b], out_hbm.at[pl.ds(off, _CHUNK)],
                                   sw[b])

    def gth(b, k):
      return pltpu.make_async_copy(
          table_sh.at[idxs[b].at[pl.ds(k * _IDX_DMA, _IDX_DMA)]],
          rows[b].at[pl.ds(k * _IDX_DMA, _IDX_DMA)],
          sg[b])

    vin(0, 0).start()
    vin(1, 1).start()

    def pair_body(i, carry):
      for b in range(2):
        c = 2 * i + b
        vin(c, b).wait()
        for g in range(_CHUNK // 16):
          gb = g * 16
          w = vals[b][pl.ds(gb, 16)]
          d0 = w & 255
          d1 = (w >> 8) & 255
          d2 = (w >> 16) & 255
          d3 = w >> 24
          idxs[b][pl.ds(gb, 16)] = d0 + d1 * 6 + d2 * 36 + d3 * 216
        @pl.when(c + 2 < _N_CHUNKS)
        def _prefetch():
          vin(c + 2, b).start()

        @pl.when(c >= 2)
        def _drain_prev_write():
          wout(c - 2, b).wait()
        for k in range(_N_GATHERS):
          gth(b, k).start()
        for k in range(_N_GATHERS):
          gth(b, k).wait()
        wout(c, b).start()
      return carry

    lax.fori_loop(0, _N_CHUNKS // 2, pair_body, 0)
    wout(_N_CHUNKS - 2, 0).wait()
    wout(_N_CHUNKS - 1, 1).wait()

  return lookup


_LOOKUP = _make_kernel()


def kernel(valences, embed_table, device):
  val_packed = lax.bitcast_convert_type(
      valences.reshape(_B, _NUM_TYPES).astype(jnp.int8), jnp.int32)
  out = _LOOKUP(val_packed, embed_table)
  return out.reshape(_BATCH, _ATOMS, _EMBED)


# speedup vs baseline: 8.8927x; 1.2774x over previous
import functools

import jax
import jax.numpy as jnp
from jax import lax
from jax.experimental import pallas as pl
from jax.experimental.pallas import tpu as pltpu
from jax.experimental.pallas import tpu_sc as plsc

_MAX_VALENCE = 6
_NUM_TYPES = 4
_VOCAB = _MAX_VALENCE ** _NUM_TYPES
_EMBED = 64
_BATCH = 16384
_ATOMS = 50
_B = _BATCH * _ATOMS

_NC = 2
_NS = 16
_NW = _NC * _NS
_BATCH_PER_W = _BATCH // _NW
_CHUNK_B = 8
_CHUNK = _CHUNK_B * _ATOMS
_N_CHUNKS = _BATCH_PER_W // _CHUNK_B
_GATHERS = (128, 128, 128, 16)


def _make_kernel():
  mesh = plsc.VectorSubcoreMesh(core_axis_name="c", subcore_axis_name="s")

  @functools.partial(
      pl.kernel,
      mesh=mesh,
      compiler_params=pltpu.CompilerParams(use_tc_tiling_on_sc=True),
      out_type=jax.ShapeDtypeStruct((_BATCH, _ATOMS, _EMBED), jnp.float32),
      scratch_types=[
          pltpu.VMEM((_CHUNK,), jnp.int32),
          pltpu.VMEM((_CHUNK,), jnp.int32),
          pltpu.VMEM((_CHUNK,), jnp.int32),
          pltpu.VMEM((_CHUNK,), jnp.int32),
          pltpu.VMEM((_CHUNK, _EMBED), jnp.float32),
          pltpu.VMEM((_CHUNK, _EMBED), jnp.float32),
          pltpu.VMEM_SHARED((_VOCAB, _EMBED), jnp.float32),
          pltpu.SemaphoreType.DMA,
          pltpu.SemaphoreType.DMA,
          pltpu.SemaphoreType.DMA,
          pltpu.SemaphoreType.DMA,
          pltpu.SemaphoreType.DMA,
          pltpu.SemaphoreType.DMA,
      ],
  )
  def lookup(val_hbm, table_hbm, out_hbm,
             val0, val1, idx0, idx1, rows0, rows1, table_sh,
             sv0, sv1, sg0, sg1, sw0, sw1):
    sid = lax.axis_index("s")
    wid = sid * _NC + lax.axis_index("c")

    @pl.when(sid == 0)
    def _stage_table():
      pltpu.sync_copy(table_hbm, table_sh)

    plsc.subcore_barrier()

    vals = (val0, val1)
    idxs = (idx0, idx1)
    rows = (rows0, rows1)
    sv = (sv0, sv1)
    sg = (sg0, sg1)
    sw = (sw0, sw1)

    def vin(c, b):
      off = (wid * _BATCH_PER_W + c * _CHUNK_B) * _ATOMS
      return pltpu.make_async_copy(val_hbm.at[pl.ds(off, _CHUNK)], vals[b],
                                   sv[b])

    def wouts(c, b):
      b0 = wid * _BATCH_PER_W + c * _CHUNK_B
      return [
          pltpu.make_async_copy(rows[b].at[pl.ds(i * _ATOMS, _ATOMS)],
                                out_hbm.at[b0 + i], sw[b])
          for i in range(_CHUNK_B)
      ]

    def gths(b):
      copies = []
      off = 0
      for n in _GATHERS:
        copies.append(pltpu.make_async_copy(
            table_sh.at[idxs[b].at[pl.ds(off, n)]],
            rows[b].at[pl.ds(off, n)],
            sg[b]))
        off += n
      return copies

    vin(0, 0).start()
    vin(1, 1).start()

    def pair_body(i, carry):
      for b in range(2):
        c = 2 * i + b
        vin(c, b).wait()
        for g in range(_CHUNK // 16):
          gb = g * 16
          w = vals[b][pl.ds(gb, 16)]
          d0 = w & 255
          d1 = (w >> 8) & 255
          d2 = (w >> 16) & 255
          d3 = w >> 24
          idxs[b][pl.ds(gb, 16)] = d0 + d1 * 6 + d2 * 36 + d3 * 216

        @pl.when(c + 2 < _N_CHUNKS)
        def _prefetch():
          vin(c + 2, b).start()

        @pl.when(c >= 2)
        def _drain_prev_write():
          for w_ in wouts(c - 2, b):
            w_.wait()

        for g_ in gths(b):
          g_.start()
        for g_ in gths(b):
          g_.wait()
        for w_ in wouts(c, b):
          w_.start()
      return carry

    lax.fori_loop(0, _N_CHUNKS // 2, pair_body, 0)
    for w_ in wouts(_N_CHUNKS - 2, 0):
      w_.wait()
    for w_ in wouts(_N_CHUNKS - 1, 1):
      w_.wait()

  return lookup


_LOOKUP = _make_kernel()


def kernel(valences, embed_table, device):
  val_packed = lax.bitcast_convert_type(
      valences.reshape(_B, _NUM_TYPES).astype(jnp.int8), jnp.int32)
  return _LOOKUP(val_packed, embed_table)


# speedup vs baseline: 9.0043x; 1.0126x over previous
import functools

import jax
import jax.numpy as jnp
from jax import lax
from jax.experimental import pallas as pl
from jax.experimental.pallas import tpu as pltpu
from jax.experimental.pallas import tpu_sc as plsc

_MAX_VALENCE = 6
_NUM_TYPES = 4
_VOCAB = _MAX_VALENCE ** _NUM_TYPES
_EMBED = 64
_BATCH = 16384
_ATOMS = 50
_B = _BATCH * _ATOMS

_NC = 2
_NS = 16
_NW = _NC * _NS
_BATCH_PER_W = _BATCH // _NW
_CHUNK_B = 8
_CHUNK = _CHUNK_B * _ATOMS
_N_CHUNKS = _BATCH_PER_W // _CHUNK_B
_GATHERS = (128, 128, 128, 16)


def _make_kernel():
  mesh = plsc.VectorSubcoreMesh(core_axis_name="c", subcore_axis_name="s")

  @functools.partial(
      pl.kernel,
      mesh=mesh,
      compiler_params=pltpu.CompilerParams(use_tc_tiling_on_sc=True),
      out_type=jax.ShapeDtypeStruct((_BATCH, _ATOMS, _EMBED), jnp.float32),
      scratch_types=[
          pltpu.VMEM((_CHUNK,), jnp.int32),
          pltpu.VMEM((_CHUNK,), jnp.int32),
          pltpu.VMEM((_CHUNK,), jnp.int32),
          pltpu.VMEM((_CHUNK,), jnp.int32),
          pltpu.VMEM((_CHUNK, _EMBED), jnp.float32),
          pltpu.VMEM((_CHUNK, _EMBED), jnp.float32),
          pltpu.VMEM_SHARED((_VOCAB, _EMBED), jnp.float32),
          pltpu.SemaphoreType.DMA,
          pltpu.SemaphoreType.DMA,
          pltpu.SemaphoreType.DMA,
          pltpu.SemaphoreType.DMA,
          pltpu.SemaphoreType.DMA,
          pltpu.SemaphoreType.DMA,
      ],
  )
  def lookup(val_hbm, table_hbm, out_hbm,
             val0, val1, idx0, idx1, rows0, rows1, table_sh,
             sv0, sv1, sg0, sg1, sw0, sw1):
    sid = lax.axis_index("s")
    wid = sid * _NC + lax.axis_index("c")

    @pl.when(sid == 0)
    def _stage_table():
      pltpu.sync_copy(table_hbm, table_sh)

    plsc.subcore_barrier()

    vals = (val0, val1)
    idxs = (idx0, idx1)
    rows = (rows0, rows1)
    sv = (sv0, sv1)
    sg = (sg0, sg1)
    sw = (sw0, sw1)

    def vin(c, b):
      off = (wid * _BATCH_PER_W + c * _CHUNK_B) * _ATOMS
      return pltpu.make_async_copy(val_hbm.at[pl.ds(off, _CHUNK)], vals[b],
                                   sv[b])

    def wouts(c, b):
      b0 = wid * _BATCH_PER_W + c * _CHUNK_B
      return [
          pltpu.make_async_copy(rows[b].at[pl.ds(i * _ATOMS, _ATOMS)],
                                out_hbm.at[b0 + i], sw[b])
          for i in range(_CHUNK_B)
      ]

    def gths(b):
      copies = []
      off = 0
      for n in _GATHERS:
        copies.append(pltpu.make_async_copy(
            table_sh.at[idxs[b].at[pl.ds(off, n)]],
            rows[b].at[pl.ds(off, n)],
            sg[b]))
        off += n
      return copies

    vin(0, 0).start()
    vin(1, 1).start()

    def pair_body(i, carry):
      for b in range(2):
        c = 2 * i + b
        vin(c, b).wait()
        for g in range(_CHUNK // 16):
          gb = g * 16
          w = vals[b][pl.ds(gb, 16)]
          d0 = w & 255
          d1 = (w >> 8) & 255
          d2 = (w >> 16) & 255
          d3 = w >> 24
          idxs[b][pl.ds(gb, 16)] = d0 + d1 * 6 + d2 * 36 + d3 * 216

        @pl.when(c + 2 < _N_CHUNKS)
        def _prefetch():
          vin(c + 2, b).start()

        @pl.when(c >= 2)
        def _drain_prev_write():
          for w_ in wouts(c - 2, b):
            w_.wait()

        for g_ in gths(b):
          g_.start()
        for g_ in gths(b):
          g_.wait()
        for w_ in wouts(c, b):
          w_.start()
      return carry

    lax.fori_loop(0, _N_CHUNKS // 2, pair_body, 0)
    for w_ in wouts(_N_CHUNKS - 2, 0):
      w_.wait()
    for w_ in wouts(_N_CHUNKS - 1, 1):
      w_.wait()

  return lookup


_LOOKUP = _make_kernel()

_TB = 128


def _transpose_body(x_ref, o_ref):
  for a in range(_ATOMS):
    o_ref[a] = jnp.transpose(x_ref[:, a, :], (1, 0))


_T2 = pl.pallas_call(
    _transpose_body,
    grid=(_BATCH // _TB,),
    in_specs=[pl.BlockSpec((_TB, _ATOMS, _EMBED), lambda i: (i, 0, 0))],
    out_specs=pl.BlockSpec((_ATOMS, _EMBED, _TB), lambda i: (0, 0, i)),
    out_shape=jax.ShapeDtypeStruct((_ATOMS, _EMBED, _BATCH), jnp.float32),
)


def kernel(valences, embed_table, device):
  val_packed = lax.bitcast_convert_type(
      valences.reshape(_B, _NUM_TYPES).astype(jnp.int8), jnp.int32)
  rows = _LOOKUP(val_packed, embed_table)
  return jnp.transpose(_T2(rows), (2, 0, 1))


# speedup vs baseline: 9.4368x; 1.0480x over previous
import functools

import jax
import jax.numpy as jnp
from jax import lax
from jax.experimental import pallas as pl
from jax.experimental.pallas import tpu as pltpu
from jax.experimental.pallas import tpu_sc as plsc

_MAX_VALENCE = 6
_NUM_TYPES = 4
_VOCAB = _MAX_VALENCE ** _NUM_TYPES
_EMBED = 64
_BATCH = 16384
_ATOMS = 50
_B = _BATCH * _ATOMS

_NC = 2
_NS = 16
_NW = _NC * _NS
_HALVES = 2
_BATCH_H = _BATCH // _HALVES
_BATCH_PER_W = _BATCH_H // _NW
_CHUNK_B = 8
_CHUNK = _CHUNK_B * _ATOMS
_N_CHUNKS = _BATCH_PER_W // _CHUNK_B
_GATHERS = (128, 128, 128, 16)


def _make_kernel():
  mesh = plsc.VectorSubcoreMesh(core_axis_name="c", subcore_axis_name="s")

  @functools.partial(
      pl.kernel,
      mesh=mesh,
      compiler_params=pltpu.CompilerParams(use_tc_tiling_on_sc=True),
      out_type=jax.ShapeDtypeStruct((_BATCH_H, _ATOMS, _EMBED), jnp.float32),
      scratch_types=[
          pltpu.VMEM((_CHUNK,), jnp.int32),
          pltpu.VMEM((_CHUNK,), jnp.int32),
          pltpu.VMEM((_CHUNK,), jnp.int32),
          pltpu.VMEM((_CHUNK,), jnp.int32),
          pltpu.VMEM((_CHUNK, _EMBED), jnp.float32),
          pltpu.VMEM((_CHUNK, _EMBED), jnp.float32),
          pltpu.VMEM_SHARED((_VOCAB, _EMBED), jnp.float32),
          pltpu.SemaphoreType.DMA,
          pltpu.SemaphoreType.DMA,
          pltpu.SemaphoreType.DMA,
          pltpu.SemaphoreType.DMA,
          pltpu.SemaphoreType.DMA,
          pltpu.SemaphoreType.DMA,
      ],
  )
  def lookup(val_hbm, table_hbm, out_hbm,
             val0, val1, idx0, idx1, rows0, rows1, table_sh,
             sv0, sv1, sg0, sg1, sw0, sw1):
    sid = lax.axis_index("s")
    wid = sid * _NC + lax.axis_index("c")

    @pl.when(sid == 0)
    def _stage_table():
      pltpu.sync_copy(table_hbm, table_sh)

    plsc.subcore_barrier()

    vals = (val0, val1)
    idxs = (idx0, idx1)
    rows = (rows0, rows1)
    sv = (sv0, sv1)
    sg = (sg0, sg1)
    sw = (sw0, sw1)

    def vin(c, b):
      off = (wid * _BATCH_PER_W + c * _CHUNK_B) * _ATOMS
      return pltpu.make_async_copy(val_hbm.at[pl.ds(off, _CHUNK)], vals[b],
                                   sv[b])

    def wouts(c, b):
      b0 = wid * _BATCH_PER_W + c * _CHUNK_B
      return [
          pltpu.make_async_copy(rows[b].at[pl.ds(i * _ATOMS, _ATOMS)],
                                out_hbm.at[b0 + i], sw[b])
          for i in range(_CHUNK_B)
      ]

    def gths(b):
      copies = []
      off = 0
      for n in _GATHERS:
        copies.append(pltpu.make_async_copy(
            table_sh.at[idxs[b].at[pl.ds(off, n)]],
            rows[b].at[pl.ds(off, n)],
            sg[b]))
        off += n
      return copies

    vin(0, 0).start()
    vin(1, 1).start()

    def pair_body(i, carry):
      for b in range(2):
        c = 2 * i + b
        vin(c, b).wait()
        for g in range(_CHUNK // 16):
          gb = g * 16
          w = vals[b][pl.ds(gb, 16)]
          d0 = w & 255
          d1 = (w >> 8) & 255
          d2 = (w >> 16) & 255
          d3 = w >> 24
          idxs[b][pl.ds(gb, 16)] = d0 + d1 * 6 + d2 * 36 + d3 * 216

        @pl.when(c + 2 < _N_CHUNKS)
        def _prefetch():
          vin(c + 2, b).start()

        @pl.when(c >= 2)
        def _drain_prev_write():
          for w_ in wouts(c - 2, b):
            w_.wait()

        for g_ in gths(b):
          g_.start()
        for g_ in gths(b):
          g_.wait()
        for w_ in wouts(c, b):
          w_.start()
      return carry

    lax.fori_loop(0, _N_CHUNKS // 2, pair_body, 0)
    for w_ in wouts(_N_CHUNKS - 2, 0):
      w_.wait()
    for w_ in wouts(_N_CHUNKS - 1, 1):
      w_.wait()

  return lookup


_LOOKUP = _make_kernel()

_TB = 128
_NTB = _BATCH_H // _TB


def _transpose_body(x_ref, o_ref):
  for a in range(_ATOMS):
    o_ref[a] = jnp.transpose(x_ref[:, a, :], (1, 0))


def _transpose_body_alias(x_ref, prev_ref, o_ref):
  del prev_ref
  for a in range(_ATOMS):
    o_ref[a] = jnp.transpose(x_ref[:, a, :], (1, 0))


_T2A = pl.pallas_call(
    _transpose_body,
    grid=(_NTB,),
    in_specs=[pl.BlockSpec((_TB, _ATOMS, _EMBED), lambda i: (i, 0, 0))],
    out_specs=pl.BlockSpec((_ATOMS, _EMBED, _TB), lambda i: (0, 0, i)),
    out_shape=jax.ShapeDtypeStruct((_ATOMS, _EMBED, _BATCH), jnp.float32),
)

_T2B = pl.pallas_call(
    _transpose_body_alias,
    grid=(_NTB,),
    in_specs=[
        pl.BlockSpec((_TB, _ATOMS, _EMBED), lambda i: (i, 0, 0)),
        pl.BlockSpec(memory_space=pl.ANY),
    ],
    out_specs=pl.BlockSpec((_ATOMS, _EMBED, _TB), lambda i: (0, 0, i + _NTB)),
    out_shape=jax.ShapeDtypeStruct((_ATOMS, _EMBED, _BATCH), jnp.float32),
    input_output_aliases={1: 0},
)


def kernel(valences, embed_table, device):
  val_packed = lax.bitcast_convert_type(
      valences.reshape(_B, _NUM_TYPES).astype(jnp.int8), jnp.int32)
  h0 = _LOOKUP(val_packed[: _BATCH_H * _ATOMS], embed_table)
  h1 = _LOOKUP(val_packed[_BATCH_H * _ATOMS:], embed_table)
  t = _T2B(h1, _T2A(h0))
  return jnp.transpose(t, (2, 0, 1))


# speedup vs baseline: 9.8082x; 1.0394x over previous
import functools

import jax
import jax.numpy as jnp
from jax import lax
from jax.experimental import pallas as pl
from jax.experimental.pallas import tpu as pltpu
from jax.experimental.pallas import tpu_sc as plsc

_MAX_VALENCE = 6
_NUM_TYPES = 4
_VOCAB = _MAX_VALENCE ** _NUM_TYPES
_EMBED = 64
_BATCH = 16384
_ATOMS = 50
_B = _BATCH * _ATOMS

_NC = 2
_NS = 16
_NW = _NC * _NS
_HALVES = 4
_BATCH_H = _BATCH // _HALVES
_BATCH_PER_W = _BATCH_H // _NW
_CHUNK_B = 8
_CHUNK = _CHUNK_B * _ATOMS
_N_CHUNKS = _BATCH_PER_W // _CHUNK_B
_GATHERS = (128, 128, 128, 16)


def _make_kernel():
  mesh = plsc.VectorSubcoreMesh(core_axis_name="c", subcore_axis_name="s")

  @functools.partial(
      pl.kernel,
      mesh=mesh,
      compiler_params=pltpu.CompilerParams(use_tc_tiling_on_sc=True),
      out_type=jax.ShapeDtypeStruct((_BATCH_H, _ATOMS, _EMBED), jnp.float32),
      scratch_types=[
          pltpu.VMEM((_CHUNK,), jnp.int32),
          pltpu.VMEM((_CHUNK,), jnp.int32),
          pltpu.VMEM((_CHUNK,), jnp.int32),
          pltpu.VMEM((_CHUNK,), jnp.int32),
          pltpu.VMEM((_CHUNK, _EMBED), jnp.float32),
          pltpu.VMEM((_CHUNK, _EMBED), jnp.float32),
          pltpu.VMEM_SHARED((_VOCAB, _EMBED), jnp.float32),
          pltpu.SemaphoreType.DMA,
          pltpu.SemaphoreType.DMA,
          pltpu.SemaphoreType.DMA,
          pltpu.SemaphoreType.DMA,
          pltpu.SemaphoreType.DMA,
          pltpu.SemaphoreType.DMA,
      ],
  )
  def lookup(val_hbm, table_hbm, out_hbm,
             val0, val1, idx0, idx1, rows0, rows1, table_sh,
             sv0, sv1, sg0, sg1, sw0, sw1):
    sid = lax.axis_index("s")
    wid = sid * _NC + lax.axis_index("c")

    @pl.when(sid == 0)
    def _stage_table():
      pltpu.sync_copy(table_hbm, table_sh)

    plsc.subcore_barrier()

    vals = (val0, val1)
    idxs = (idx0, idx1)
    rows = (rows0, rows1)
    sv = (sv0, sv1)
    sg = (sg0, sg1)
    sw = (sw0, sw1)

    def vin(c, b):
      off = (wid * _BATCH_PER_W + c * _CHUNK_B) * _ATOMS
      return pltpu.make_async_copy(val_hbm.at[pl.ds(off, _CHUNK)], vals[b],
                                   sv[b])

    def wouts(c, b):
      b0 = wid * _BATCH_PER_W + c * _CHUNK_B
      return [
          pltpu.make_async_copy(rows[b].at[pl.ds(i * _ATOMS, _ATOMS)],
                                out_hbm.at[b0 + i], sw[b])
          for i in range(_CHUNK_B)
      ]

    def gths(b):
      copies = []
      off = 0
      for n in _GATHERS:
        copies.append(pltpu.make_async_copy(
            table_sh.at[idxs[b].at[pl.ds(off, n)]],
            rows[b].at[pl.ds(off, n)],
            sg[b]))
        off += n
      return copies

    vin(0, 0).start()
    vin(1, 1).start()

    def pair_body(i, carry):
      for b in range(2):
        c = 2 * i + b
        vin(c, b).wait()
        for g in range(_CHUNK // 16):
          gb = g * 16
          w = vals[b][pl.ds(gb, 16)]
          d0 = w & 255
          d1 = (w >> 8) & 255
          d2 = (w >> 16) & 255
          d3 = w >> 24
          idxs[b][pl.ds(gb, 16)] = d0 + d1 * 6 + d2 * 36 + d3 * 216

        @pl.when(c + 2 < _N_CHUNKS)
        def _prefetch():
          vin(c + 2, b).start()

        @pl.when(c >= 2)
        def _drain_prev_write():
          for w_ in wouts(c - 2, b):
            w_.wait()

        for g_ in gths(b):
          g_.start()
        for g_ in gths(b):
          g_.wait()
        for w_ in wouts(c, b):
          w_.start()
      return carry

    lax.fori_loop(0, _N_CHUNKS // 2, pair_body, 0)
    for w_ in wouts(_N_CHUNKS - 2, 0):
      w_.wait()
    for w_ in wouts(_N_CHUNKS - 1, 1):
      w_.wait()

  return lookup


_LOOKUP = _make_kernel()

_TB = 128
_NTB = _BATCH_H // _TB


def _transpose_body(x_ref, o_ref):
  for a in range(_ATOMS):
    o_ref[a] = jnp.transpose(x_ref[:, a, :], (1, 0))


def _transpose_body_alias(x_ref, prev_ref, o_ref):
  del prev_ref
  for a in range(_ATOMS):
    o_ref[a] = jnp.transpose(x_ref[:, a, :], (1, 0))


def _make_t2(part):
  off = part * _NTB
  if part == 0:
    return pl.pallas_call(
        _transpose_body,
        grid=(_NTB,),
        in_specs=[pl.BlockSpec((_TB, _ATOMS, _EMBED), lambda i: (i, 0, 0))],
        out_specs=pl.BlockSpec((_ATOMS, _EMBED, _TB), lambda i: (0, 0, i)),
        out_shape=jax.ShapeDtypeStruct((_ATOMS, _EMBED, _BATCH), jnp.float32),
    )
  return pl.pallas_call(
      _transpose_body_alias,
      grid=(_NTB,),
      in_specs=[
          pl.BlockSpec((_TB, _ATOMS, _EMBED), lambda i: (i, 0, 0)),
          pl.BlockSpec(memory_space=pl.ANY),
      ],
      out_specs=pl.BlockSpec((_ATOMS, _EMBED, _TB),
                             lambda i, off=off: (0, 0, i + off)),
      out_shape=jax.ShapeDtypeStruct((_ATOMS, _EMBED, _BATCH), jnp.float32),
      input_output_aliases={1: 0},
  )


_T2 = [_make_t2(p) for p in range(_HALVES)]


def kernel(valences, embed_table, device):
  val_packed = lax.bitcast_convert_type(
      valences.reshape(_B, _NUM_TYPES).astype(jnp.int8), jnp.int32)
  n = _BATCH_H * _ATOMS
  parts = [_LOOKUP(val_packed[p * n:(p + 1) * n], embed_table)
           for p in range(_HALVES)]
  t = _T2[0](parts[0])
  for p in range(1, _HALVES):
    t = _T2[p](parts[p], t)
  return jnp.transpose(t, (2, 0, 1))
